# baseline (speedup 1.0000x reference)
import functools

import jax
import jax.numpy as jnp
import numpy as np
from jax import lax
from jax.experimental import pallas as pl

_GRID_SIZES = (19, 38, 76)
_INP_DIM = 608.0
_NUM_ANCH = 3
_L_COORD = 1.0
_L_OBJ = 5.0
_L_NOOBJ = 0.5
_B, _T = 16, 8
_N = 3 * (19 * 19 + 38 * 38 + 76 * 76)
_C = 85
_NUM_CLASSES = 80
_M = _B * _T
_K = 9

_TN = 2048
_NSTEP = -(-_N // _TN)

_AW = (116., 156., 373., 30., 62., 59., 10., 16., 33.)
_AH = (90., 198., 326., 61., 45., 119., 13., 30., 23.)


def _a_body(confT_ref, cx_ref, cy_ref, cw_ref, ch_ref, candis_ref, tb_ref,
            noobj_ref, cols_ref, ti_ref):
    j = pl.program_id(0)
    conf = confT_ref[0]
    lane = lax.broadcasted_iota(jnp.int32, (_B, _TN), 1) + j * _TN
    x = jnp.where(lane < _N, 1.0 - conf, 1.0)
    s = jnp.sum(jnp.maximum(jnp.log(x), -100.0))

    @pl.when(j == 0)
    def _init():
        noobj_ref[...] = s.reshape(1, 1)
        cx, cy = cx_ref[...], cy_ref[...]
        cw, ch = cw_ref[...], ch_ref[...]
        x1, y1 = cx - cw / 2.0, cy - ch / 2.0
        x2, y2 = cx + cw / 2.0, cy + ch / 2.0
        tb = tb_ref[...]
        ix1 = jnp.maximum(tb[:, 0:1], x1)
        iy1 = jnp.maximum(tb[:, 1:2], y1)
        ix2 = jnp.minimum(tb[:, 2:3], x2)
        iy2 = jnp.minimum(tb[:, 3:4], y2)
        inter = jnp.maximum(ix2 - ix1, 0.0) * jnp.maximum(iy2 - iy1, 0.0)
        a1 = (tb[:, 2:3] - tb[:, 0:1]) * (tb[:, 3:4] - tb[:, 1:2])
        a2 = (x2 - x1) * (y2 - y1)
        iou = inter / (a1 + a2 - inter + 1e-16)
        kio = lax.broadcasted_iota(jnp.int32, (_M, _K), 1)
        mx = jnp.max(iou, axis=1, keepdims=True)
        ti = jnp.min(jnp.where(iou == mx, kio, _K), axis=1, keepdims=True)
        cols_ref[...] = jnp.sum(
            jnp.where(kio == ti, candis_ref[...], 0), axis=1, keepdims=True)
        ti_ref[...] = ti

    @pl.when(j > 0)
    def _acc():
        noobj_ref[...] += s.reshape(1, 1)


def _b_body(noobj_ref, ids_c_ref, ids_r_ref, ti_ref, cls_ref, conf_ref,
            csel_ref, boxes_ref, ycls_ref, out_ref):
    ti = ti_ref[...]
    gidx = ti // _NUM_ANCH
    gf = jnp.where(gidx == 0, 19.0, jnp.where(gidx == 1, 38.0, 76.0))
    aw = jnp.full((_M, 1), _AW[0], jnp.float32)
    ah = jnp.full((_M, 1), _AH[0], jnp.float32)
    for k in range(1, _K):
        aw = jnp.where(ti == k, _AW[k], aw)
        ah = jnp.where(ti == k, _AH[k], ah)
    boxes = boxes_ref[...]
    bx, by = boxes[:, 0:1], boxes[:, 1:2]
    bw, bh = boxes[:, 2:3], boxes[:, 3:4]
    fx = bx * gf
    fy = by * gf
    fx = fx - jnp.floor(fx) + 1e-05
    fy = fy - jnp.floor(fy) + 1e-05
    tx = jnp.log(fx / (1.0 - fx))
    ty = jnp.log(fy / (1.0 - fy))
    tw = jnp.log(bw * _INP_DIM / aw)
    th = jnp.log(bh * _INP_DIM / ah)
    cs = csel_ref[...]
    coord_loss = _L_COORD * jnp.sum(
        (cs[:, 0:1] - tx) ** 2 + (cs[:, 1:2] - ty) ** 2
        + (cs[:, 2:3] - tw) ** 2 + (cs[:, 3:4] - th) ** 2)

    c80 = lax.broadcasted_iota(jnp.int32, (_M, _NUM_CLASSES), 1)
    tcls = jnp.where(c80 == ycls_ref[...], 1.0, 0.0)
    p = cls_ref[...]
    cls_loss = -jnp.sum(
        jnp.maximum(jnp.log(p), -100.0) * tcls
        + jnp.maximum(jnp.log(1.0 - p), -100.0) * (1.0 - tcls))

    eq = ids_c_ref[...] == ids_r_ref[...]
    lower = (lax.broadcasted_iota(jnp.int32, (_M, _M), 1)
             < lax.broadcasted_iota(jnp.int32, (_M, _M), 0))
    ndup = jnp.sum(jnp.where(eq & lower, 1.0, 0.0), axis=1, keepdims=True)
    first = jnp.where(ndup > 0.0, 0.0, 1.0)
    cg = conf_ref[...]
    logp = jnp.maximum(jnp.log(cg), -100.0)
    l1m = jnp.maximum(jnp.log(1.0 - cg), -100.0)
    noobj_total = jnp.sum(noobj_ref[...])
    conf_loss = (_L_OBJ * (-jnp.sum(logp * first))
                 + _L_NOOBJ * (-(noobj_total - jnp.sum(l1m * first))))
    out_ref[...] = (coord_loss + conf_loss + cls_loss).reshape(1, 1)


def kernel(pred_x, coord_x, y_cls, y_coord):
    f32, i32 = jnp.float32, jnp.int32
    predT = jnp.transpose(pred_x, (2, 0, 1))

    boxes = y_coord.reshape(-1, 4)
    rows = jnp.repeat(jnp.arange(_B, dtype=i32), _T)
    cand_parts = []
    base = 0
    for g in _GRID_SIZES:
        idx0 = base + ((jnp.floor(boxes[:, 1] * g)
                        + jnp.floor(boxes[:, 0] * g) * g) * _NUM_ANCH
                       ).astype(i32)
        cand_parts.append(idx0[:, None]
                          + jnp.arange(_NUM_ANCH, dtype=i32)[None, :])
        base += g * g * _NUM_ANCH
    candis = jnp.concatenate(cand_parts, axis=1)
    tb = jnp.stack([_INP_DIM * (boxes[:, 0] - boxes[:, 2] / 2),
                    _INP_DIM * (boxes[:, 1] - boxes[:, 3] / 2),
                    _INP_DIM * (boxes[:, 0] + boxes[:, 2] / 2),
                    _INP_DIM * (boxes[:, 1] + boxes[:, 3] / 2)], axis=1)

    ci_cand = jnp.broadcast_to(
        candis.reshape(1, _B, _T * _K), (_C, _B, _T * _K))
    gat = jnp.take_along_axis(predT, ci_cand, axis=2)[:4]
    cxc = gat[0].reshape(_M, _K)
    cyc = gat[1].reshape(_M, _K)
    cwc = gat[2].reshape(_M, _K)
    chc = gat[3].reshape(_M, _K)

    noobj, cols2, ti2 = pl.pallas_call(
        _a_body,
        grid=(_NSTEP,),
        in_specs=[
            pl.BlockSpec((1, _B, _TN), lambda j: (4, 0, j)),
            pl.BlockSpec((_M, _K), lambda j: (0, 0)),
            pl.BlockSpec((_M, _K), lambda j: (0, 0)),
            pl.BlockSpec((_M, _K), lambda j: (0, 0)),
            pl.BlockSpec((_M, _K), lambda j: (0, 0)),
            pl.BlockSpec((_M, _K), lambda j: (0, 0)),
            pl.BlockSpec((_M, 4), lambda j: (0, 0)),
        ],
        out_specs=[
            pl.BlockSpec((1, 1), lambda j: (0, 0)),
            pl.BlockSpec((_M, 1), lambda j: (0, 0)),
            pl.BlockSpec((_M, 1), lambda j: (0, 0)),
        ],
        out_shape=[
            jax.ShapeDtypeStruct((1, 1), f32),
            jax.ShapeDtypeStruct((_M, 1), i32),
            jax.ShapeDtypeStruct((_M, 1), i32),
        ],
    )(predT, cxc, cyc, cwc, chc, candis, tb)

    cols = cols2.reshape(-1)
    ids = rows * _N + cols

    ci_sel = jnp.broadcast_to(cols.reshape(1, _B, _T), (_C, _B, _T))
    selT = jnp.take_along_axis(predT, ci_sel, axis=2)
    sel = selT.transpose(1, 2, 0).reshape(_M, _C)
    ci_co = jnp.broadcast_to(cols.reshape(_B, _T, 1), (_B, _T, 4))
    csel = jnp.take_along_axis(coord_x, ci_co, axis=1).reshape(_M, 4)

    out = pl.pallas_call(
        _b_body,
        in_specs=[
            pl.BlockSpec((1, 1), lambda: (0, 0)),
            pl.BlockSpec((_M, 1), lambda: (0, 0)),
            pl.BlockSpec((1, _M), lambda: (0, 0)),
            pl.BlockSpec((_M, 1), lambda: (0, 0)),
            pl.BlockSpec((_M, _NUM_CLASSES), lambda: (0, 0)),
            pl.BlockSpec((_M, 1), lambda: (0, 0)),
            pl.BlockSpec((_M, 4), lambda: (0, 0)),
            pl.BlockSpec((_M, 4), lambda: (0, 0)),
            pl.BlockSpec((_M, 1), lambda: (0, 0)),
        ],
        out_specs=pl.BlockSpec((1, 1), lambda: (0, 0)),
        out_shape=jax.ShapeDtypeStruct((1, 1), f32),
    )(noobj, ids.reshape(_M, 1), ids.reshape(1, _M), ti2,
      sel[:, 5:], sel[:, 4:5], csel, boxes,
      y_cls.reshape(_M, 1))

    return out.reshape(())


# speedup vs baseline: 1.0247x; 1.0247x over previous
import functools

import jax
import jax.numpy as jnp
import numpy as np
from jax import lax
from jax.experimental import pallas as pl

_GRID_SIZES = (19, 38, 76)
_INP_DIM = 608.0
_NUM_ANCH = 3
_L_COORD = 1.0
_L_OBJ = 5.0
_L_NOOBJ = 0.5
_B, _T = 16, 8
_N = 3 * (19 * 19 + 38 * 38 + 76 * 76)
_C = 85
_NUM_CLASSES = 80
_M = _B * _T
_K = 9

_TN = 2048
_NSTEP = -(-_N // _TN)

_AW = (116., 156., 373., 30., 62., 59., 10., 16., 33.)
_AH = (90., 198., 326., 61., 45., 119., 13., 30., 23.)


def _a_body(confT_ref, cx_ref, cy_ref, cw_ref, ch_ref, candis_ref, tb_ref,
            noobj_ref, cols_ref, ti_ref):
    j = pl.program_id(0)
    conf = confT_ref[0]
    lane = lax.broadcasted_iota(jnp.int32, (_B, _TN), 1) + j * _TN
    x = jnp.where(lane < _N, 1.0 - conf, 1.0)
    s = jnp.sum(jnp.maximum(jnp.log(x), -100.0))

    @pl.when(j == 0)
    def _init():
        noobj_ref[...] = s.reshape(1, 1)
        cx, cy = cx_ref[...], cy_ref[...]
        cw, ch = cw_ref[...], ch_ref[...]
        x1, y1 = cx - cw / 2.0, cy - ch / 2.0
        x2, y2 = cx + cw / 2.0, cy + ch / 2.0
        tb = tb_ref[...]
        ix1 = jnp.maximum(tb[:, 0:1], x1)
        iy1 = jnp.maximum(tb[:, 1:2], y1)
        ix2 = jnp.minimum(tb[:, 2:3], x2)
        iy2 = jnp.minimum(tb[:, 3:4], y2)
        inter = jnp.maximum(ix2 - ix1, 0.0) * jnp.maximum(iy2 - iy1, 0.0)
        a1 = (tb[:, 2:3] - tb[:, 0:1]) * (tb[:, 3:4] - tb[:, 1:2])
        a2 = (x2 - x1) * (y2 - y1)
        iou = inter / (a1 + a2 - inter + 1e-16)
        kio = lax.broadcasted_iota(jnp.int32, (_M, _K), 1)
        mx = jnp.max(iou, axis=1, keepdims=True)
        ti = jnp.min(jnp.where(iou == mx, kio, _K), axis=1, keepdims=True)
        cols_ref[...] = jnp.sum(
            jnp.where(kio == ti, candis_ref[...], 0), axis=1, keepdims=True)
        ti_ref[...] = ti

    @pl.when(j > 0)
    def _acc():
        noobj_ref[...] += s.reshape(1, 1)


def _b_body(noobj_ref, ids_c_ref, ids_r_ref, ti_ref, cls_ref, conf_ref,
            csel_ref, boxes_ref, ycls_ref, out_ref):
    ti = ti_ref[...]
    gidx = ti // _NUM_ANCH
    gf = jnp.where(gidx == 0, 19.0, jnp.where(gidx == 1, 38.0, 76.0))
    aw = jnp.full((_M, 1), _AW[0], jnp.float32)
    ah = jnp.full((_M, 1), _AH[0], jnp.float32)
    for k in range(1, _K):
        aw = jnp.where(ti == k, _AW[k], aw)
        ah = jnp.where(ti == k, _AH[k], ah)
    boxes = boxes_ref[...]
    bx, by = boxes[:, 0:1], boxes[:, 1:2]
    bw, bh = boxes[:, 2:3], boxes[:, 3:4]
    fx = bx * gf
    fy = by * gf
    fx = fx - jnp.floor(fx) + 1e-05
    fy = fy - jnp.floor(fy) + 1e-05
    tx = jnp.log(fx / (1.0 - fx))
    ty = jnp.log(fy / (1.0 - fy))
    tw = jnp.log(bw * _INP_DIM / aw)
    th = jnp.log(bh * _INP_DIM / ah)
    cs = csel_ref[...]
    coord_loss = _L_COORD * jnp.sum(
        (cs[:, 0:1] - tx) ** 2 + (cs[:, 1:2] - ty) ** 2
        + (cs[:, 2:3] - tw) ** 2 + (cs[:, 3:4] - th) ** 2)

    c80 = lax.broadcasted_iota(jnp.int32, (_M, _NUM_CLASSES), 1)
    tcls = jnp.where(c80 == ycls_ref[...], 1.0, 0.0)
    p = cls_ref[...]
    cls_loss = -jnp.sum(
        jnp.maximum(jnp.log(p), -100.0) * tcls
        + jnp.maximum(jnp.log(1.0 - p), -100.0) * (1.0 - tcls))

    eq = ids_c_ref[...] == ids_r_ref[...]
    lower = (lax.broadcasted_iota(jnp.int32, (_M, _M), 1)
             < lax.broadcasted_iota(jnp.int32, (_M, _M), 0))
    ndup = jnp.sum(jnp.where(eq & lower, 1.0, 0.0), axis=1, keepdims=True)
    first = jnp.where(ndup > 0.0, 0.0, 1.0)
    cg = conf_ref[...]
    logp = jnp.maximum(jnp.log(cg), -100.0)
    l1m = jnp.maximum(jnp.log(1.0 - cg), -100.0)
    noobj_total = jnp.sum(noobj_ref[...])
    conf_loss = (_L_OBJ * (-jnp.sum(logp * first))
                 + _L_NOOBJ * (-(noobj_total - jnp.sum(l1m * first))))
    out_ref[...] = (coord_loss + conf_loss + cls_loss).reshape(1, 1)


def kernel(pred_x, coord_x, y_cls, y_coord):
    f32, i32 = jnp.float32, jnp.int32
    predT = jnp.transpose(pred_x, (2, 0, 1))
    coordT = jnp.transpose(coord_x, (2, 0, 1))

    boxes = y_coord.reshape(-1, 4)
    rows = jnp.repeat(jnp.arange(_B, dtype=i32), _T)
    cand_parts = []
    base = 0
    for g in _GRID_SIZES:
        idx0 = base + ((jnp.floor(boxes[:, 1] * g)
                        + jnp.floor(boxes[:, 0] * g) * g) * _NUM_ANCH
                       ).astype(i32)
        cand_parts.append(idx0[:, None]
                          + jnp.arange(_NUM_ANCH, dtype=i32)[None, :])
        base += g * g * _NUM_ANCH
    candis = jnp.concatenate(cand_parts, axis=1)
    tb = jnp.stack([_INP_DIM * (boxes[:, 0] - boxes[:, 2] / 2),
                    _INP_DIM * (boxes[:, 1] - boxes[:, 3] / 2),
                    _INP_DIM * (boxes[:, 0] + boxes[:, 2] / 2),
                    _INP_DIM * (boxes[:, 1] + boxes[:, 3] / 2)], axis=1)

    ci_cand = jnp.broadcast_to(
        candis.reshape(1, _B, _T * _K), (_C, _B, _T * _K))
    gat = jnp.take_along_axis(predT, ci_cand, axis=2)[:4]
    cxc = gat[0].reshape(_M, _K)
    cyc = gat[1].reshape(_M, _K)
    cwc = gat[2].reshape(_M, _K)
    chc = gat[3].reshape(_M, _K)

    noobj, cols2, ti2 = pl.pallas_call(
        _a_body,
        grid=(_NSTEP,),
        in_specs=[
            pl.BlockSpec((1, _B, _TN), lambda j: (4, 0, j)),
            pl.BlockSpec((_M, _K), lambda j: (0, 0)),
            pl.BlockSpec((_M, _K), lambda j: (0, 0)),
            pl.BlockSpec((_M, _K), lambda j: (0, 0)),
            pl.BlockSpec((_M, _K), lambda j: (0, 0)),
            pl.BlockSpec((_M, _K), lambda j: (0, 0)),
            pl.BlockSpec((_M, 4), lambda j: (0, 0)),
        ],
        out_specs=[
            pl.BlockSpec((1, 1), lambda j: (0, 0)),
            pl.BlockSpec((_M, 1), lambda j: (0, 0)),
            pl.BlockSpec((_M, 1), lambda j: (0, 0)),
        ],
        out_shape=[
            jax.ShapeDtypeStruct((1, 1), f32),
            jax.ShapeDtypeStruct((_M, 1), i32),
            jax.ShapeDtypeStruct((_M, 1), i32),
        ],
    )(predT, cxc, cyc, cwc, chc, candis, tb)

    cols = cols2.reshape(-1)
    ids = rows * _N + cols

    ci_sel = jnp.broadcast_to(cols.reshape(1, _B, _T), (_C, _B, _T))
    selT = jnp.take_along_axis(predT, ci_sel, axis=2)
    sel = selT.transpose(1, 2, 0).reshape(_M, _C)
    ci_co = jnp.broadcast_to(cols.reshape(1, _B, _T), (4, _B, _T))
    csel = jnp.take_along_axis(coordT, ci_co, axis=2
                               ).transpose(1, 2, 0).reshape(_M, 4)

    out = pl.pallas_call(
        _b_body,
        in_specs=[
            pl.BlockSpec((1, 1), lambda: (0, 0)),
            pl.BlockSpec((_M, 1), lambda: (0, 0)),
            pl.BlockSpec((1, _M), lambda: (0, 0)),
            pl.BlockSpec((_M, 1), lambda: (0, 0)),
            pl.BlockSpec((_M, _NUM_CLASSES), lambda: (0, 0)),
            pl.BlockSpec((_M, 1), lambda: (0, 0)),
            pl.BlockSpec((_M, 4), lambda: (0, 0)),
            pl.BlockSpec((_M, 4), lambda: (0, 0)),
            pl.BlockSpec((_M, 1), lambda: (0, 0)),
        ],
        out_specs=pl.BlockSpec((1, 1), lambda: (0, 0)),
        out_shape=jax.ShapeDtypeStruct((1, 1), f32),
    )(noobj, ids.reshape(_M, 1), ids.reshape(1, _M), ti2,
      sel[:, 5:], sel[:, 4:5], csel, boxes,
      y_cls.reshape(_M, 1))

    return out.reshape(())
